# Initial kernel scaffold; baseline (speedup 1.0000x reference)
import jax
import jax.numpy as jnp
from jax.experimental import pallas as pl


def kernel(edge_attr, W0, W1, W2):
    raise NotImplementedError("write your pallas kernel here")


# speedup vs baseline: 1.0860x; 1.0860x over previous
import functools

import jax
import jax.numpy as jnp
from jax import lax
from jax.experimental import pallas as pl
from jax.experimental.pallas import tpu as pltpu
from jax.experimental.pallas import tpu_sc as plsc

EMB = 128
E = 320000
NW = 32
CHUNK = 128
NCHUNKS = E // CHUNK
PER_TILE = -(-NCHUNKS // NW)


def _table_body(w0_ref, w1_ref, w2_ref, t_ref):
    for i in range(4):
        for j in range(2):
            base = i * 12 + j * 6
            t_ref[base:base + 6, :] = (
                w2_ref[:, :] + w0_ref[i:i + 1, :] + w1_ref[j:j + 1, :]
            )


def _build_table(w0, w1, w2):
    return pl.pallas_call(
        _table_body,
        out_shape=jax.ShapeDtypeStruct((48, EMB), jnp.float32),
    )(w0, w1, w2)


def _sc_body(a0_hbm, a1_hbm, a2_hbm, table_hbm, out_hbm,
             a0_v, a1_v, a2_v, idx_v, rows_v, sem):
    wid = lax.axis_index("s") * 2 + lax.axis_index("c")

    def body(i, carry):
        cid = i * NW + wid

        @pl.when(cid < NCHUNKS)
        def _():
            base = cid * CHUNK
            pltpu.sync_copy(a0_hbm.at[pl.ds(base, CHUNK)], a0_v)
            pltpu.sync_copy(a1_hbm.at[pl.ds(base, CHUNK)], a1_v)
            pltpu.sync_copy(a2_hbm.at[pl.ds(base, CHUNK)], a2_v)
            for s in range(CHUNK // 16):
                sl = pl.ds(s * 16, 16)
                idx_v[sl] = a0_v[sl] * 12 + a1_v[sl] * 6 + a2_v[sl]
            pltpu.async_copy(table_hbm.at[idx_v], rows_v, sem).wait()
            pltpu.sync_copy(rows_v, out_hbm.at[pl.ds(base, CHUNK)])

        return carry

    lax.fori_loop(0, PER_TILE, body, 0)


_sc_gather = functools.partial(
    pl.kernel,
    out_type=jax.ShapeDtypeStruct((E, EMB), jnp.float32),
    mesh=plsc.VectorSubcoreMesh(core_axis_name="c", subcore_axis_name="s"),
    scratch_types=[
        pltpu.VMEM((CHUNK,), jnp.int32),
        pltpu.VMEM((CHUNK,), jnp.int32),
        pltpu.VMEM((CHUNK,), jnp.int32),
        pltpu.VMEM((CHUNK,), jnp.int32),
        pltpu.VMEM((CHUNK, EMB), jnp.float32),
        pltpu.SemaphoreType.DMA,
    ],
)(_sc_body)


@jax.jit
def kernel(edge_attr, W0, W1, W2):
    a = edge_attr.astype(jnp.int32)
    table = _build_table(W0, W1, W2)
    return _sc_gather(a[:, 0], a[:, 1], a[:, 2], table)


# speedup vs baseline: 1.0917x; 1.0052x over previous
import functools

import jax
import jax.numpy as jnp
from jax import lax
from jax.experimental import pallas as pl
from jax.experimental.pallas import tpu as pltpu
from jax.experimental.pallas import tpu_sc as plsc

EMB = 128
E = 320000
NW = 32
GATHER = 128
K = 2
SUPER = K * GATHER
NSUPER = E // SUPER
T_FULL = NSUPER // NW
NLEFT = NSUPER - T_FULL * NW


def _table_body(w0_ref, w1_ref, w2_ref, t_ref):
    for i in range(4):
        for j in range(2):
            base = i * 12 + j * 6
            t_ref[base:base + 6, :] = (
                w2_ref[:, :] + w0_ref[i:i + 1, :] + w1_ref[j:j + 1, :]
            )


def _build_table(w0, w1, w2):
    return pl.pallas_call(
        _table_body,
        out_shape=jax.ShapeDtypeStruct((48, EMB), jnp.float32),
    )(w0, w1, w2)


def _sc_body(a0_hbm, a1_hbm, a2_hbm, table_hbm, out_hbm,
             a0_v, a1_v, a2_v, idx_v, rows_v,
             a_sem0, a_sem1, g_sem0, g_sem1, o_sem0, o_sem1):
    wid = lax.axis_index("s") * 2 + lax.axis_index("c")
    a_sems = (a_sem0, a_sem1)
    g_sems = (g_sem0, g_sem1)
    o_sems = (o_sem0, o_sem1)

    def cid_of(t):
        return t * NW + wid

    def attr_copies(t, b):
        base = cid_of(t) * SUPER
        return [
            pltpu.make_async_copy(a0_hbm.at[pl.ds(base, SUPER)], a0_v.at[b], a_sems[b]),
            pltpu.make_async_copy(a1_hbm.at[pl.ds(base, SUPER)], a1_v.at[b], a_sems[b]),
            pltpu.make_async_copy(a2_hbm.at[pl.ds(base, SUPER)], a2_v.at[b], a_sems[b]),
        ]

    def issue_attr(t, b):
        for c in attr_copies(t, b):
            c.start()

    def wait_attr(t, b):
        for c in attr_copies(t, b):
            c.wait()

    def compute_codes(b):
        for k in range(K):
            for s in range(8):
                sl = pl.ds(k * GATHER + s * 16, 16)
                idx_v[b, k, pl.ds(s * 16, 16)] = (
                    a0_v[b, sl] * 12 + a1_v[b, sl] * 6 + a2_v[b, sl]
                )

    def gather_copies(b):
        return [
            pltpu.make_async_copy(
                table_hbm.at[idx_v.at[b, k]],
                rows_v.at[b, pl.ds(k * GATHER, GATHER)],
                g_sems[b],
            )
            for k in range(K)
        ]

    def issue_gather(b):
        for c in gather_copies(b):
            c.start()

    def wait_gather(b):
        for c in gather_copies(b):
            c.wait()

    def out_copy(t, b):
        base = cid_of(t) * SUPER
        return pltpu.make_async_copy(
            rows_v.at[b], out_hbm.at[pl.ds(base, SUPER)], o_sems[b])

    issue_attr(0, 0)
    issue_attr(1, 1)
    wait_attr(0, 0)
    compute_codes(0)
    issue_attr(2, 0)
    issue_gather(0)
    wait_attr(1, 1)
    compute_codes(1)
    issue_attr(3, 1)
    issue_gather(1)
    wait_gather(0)
    out_copy(0, 0).start()

    def steady(t0, j_is_static):
        wait_attr(t0, 0)
        compute_codes(0)
        if not j_is_static or t0 + 2 <= T_FULL - 1:
            issue_attr(t0 + 2, 0)
        out_copy(t0 - 2, 0).wait()
        issue_gather(0)
        wait_gather(1)
        out_copy(t0 - 1, 1).start()

        wait_attr(t0 + 1, 1)
        compute_codes(1)
        if not j_is_static or t0 + 3 <= T_FULL - 1:
            issue_attr(t0 + 3, 1)
        out_copy(t0 - 1, 1).wait()
        issue_gather(1)
        wait_gather(0)
        out_copy(t0, 0).start()

    def loop_body(j, carry):
        steady(j * 2, False)
        return carry

    lax.fori_loop(1, 18, loop_body, 0)

    steady(36, True)

    t = 38
    wait_attr(t, 0)
    compute_codes(0)
    out_copy(t - 2, 0).wait()
    issue_gather(0)
    wait_gather(1)
    out_copy(t - 1, 1).start()
    wait_gather(0)
    out_copy(t, 0).start()
    out_copy(t - 1, 1).wait()
    out_copy(t, 0).wait()

    @pl.when(wid < NLEFT)
    def _():
        t_extra = T_FULL
        issue_attr(t_extra, 0)
        wait_attr(t_extra, 0)
        compute_codes(0)
        issue_gather(0)
        wait_gather(0)
        out_copy(t_extra, 0).start()
        out_copy(t_extra, 0).wait()


_sc_gather = functools.partial(
    pl.kernel,
    out_type=jax.ShapeDtypeStruct((E, EMB), jnp.float32),
    mesh=plsc.VectorSubcoreMesh(core_axis_name="c", subcore_axis_name="s"),
    scratch_types=[
        pltpu.VMEM((2, SUPER), jnp.int32),
        pltpu.VMEM((2, SUPER), jnp.int32),
        pltpu.VMEM((2, SUPER), jnp.int32),
        pltpu.VMEM((2, K, GATHER), jnp.int32),
        pltpu.VMEM((2, SUPER, EMB), jnp.float32),
        pltpu.SemaphoreType.DMA,
        pltpu.SemaphoreType.DMA,
        pltpu.SemaphoreType.DMA,
        pltpu.SemaphoreType.DMA,
        pltpu.SemaphoreType.DMA,
        pltpu.SemaphoreType.DMA,
    ],
)(_sc_body)


@jax.jit
def kernel(edge_attr, W0, W1, W2):
    a = edge_attr.astype(jnp.int32)
    table = _build_table(W0, W1, W2)
    return _sc_gather(a[:, 0], a[:, 1], a[:, 2], table)


# speedup vs baseline: 1.3174x; 1.2068x over previous
import functools

import jax
import jax.numpy as jnp
from jax import lax
from jax.experimental import pallas as pl
from jax.experimental.pallas import tpu as pltpu
from jax.experimental.pallas import tpu_sc as plsc

EMB = 128
E = 320000
NW = 32
SUPER = 256
GROUPS = SUPER // 16
NSUPER = E // SUPER
T_FULL = NSUPER // NW
NLEFT = NSUPER - T_FULL * NW


def _table_body(w0_ref, w1_ref, w2_ref, t_ref):
    for i in range(4):
        for j in range(2):
            base = i * 12 + j * 6
            t_ref[base:base + 6, :] = (
                w2_ref[:, :] + w0_ref[i:i + 1, :] + w1_ref[j:j + 1, :]
            )


def _build_table(w0, w1, w2):
    return pl.pallas_call(
        _table_body,
        out_shape=jax.ShapeDtypeStruct((48, EMB), jnp.float32),
    )(w0, w1, w2)


def _sc_body(a0_hbm, a1_hbm, a2_hbm, table_hbm, out_hbm,
             table_v, a0_v, a1_v, a2_v, rows0_v, rows1_v,
             a_sem0, a_sem1, o_sem0, o_sem1, t_sem):
    wid = lax.axis_index("s") * 2 + lax.axis_index("c")
    a_sems = (a_sem0, a_sem1)
    o_sems = (o_sem0, o_sem1)
    rows = (rows0_v, rows1_v)

    def cid_of(t):
        return t * NW + wid

    def attr_copies(t, b):
        base = cid_of(t) * SUPER
        return [
            pltpu.make_async_copy(a0_hbm.at[pl.ds(base, SUPER)], a0_v.at[b], a_sems[b]),
            pltpu.make_async_copy(a1_hbm.at[pl.ds(base, SUPER)], a1_v.at[b], a_sems[b]),
            pltpu.make_async_copy(a2_hbm.at[pl.ds(base, SUPER)], a2_v.at[b], a_sems[b]),
        ]

    def issue_attr(t, b):
        for c in attr_copies(t, b):
            c.start()

    def wait_attr(t, b):
        for c in attr_copies(t, b):
            c.wait()

    def out_copy(t, b):
        base = cid_of(t) * SUPER * EMB
        return pltpu.make_async_copy(
            rows[b], out_hbm.at[pl.ds(base, SUPER * EMB)], o_sems[b])

    iota = lax.iota(jnp.int32, 16)
    iota128 = iota * EMB

    def expand(b):
        rb = rows[b]

        def group(g, carry):
            sl = pl.ds(g * 16, 16)
            code = a0_v[b, sl] * 12 + a1_v[b, sl] * 6 + a2_v[b, sl]
            src0 = code * EMB
            dst0 = g * (16 * EMB) + iota128
            for c in range(EMB):
                vals = plsc.load_gather(table_v, [src0 + c])
                plsc.store_scatter(rb, [dst0 + c], vals)
            return carry

        lax.fori_loop(0, GROUPS, group, 0)

    pltpu.async_copy(table_hbm, table_v, t_sem).wait()

    issue_attr(0, 0)
    issue_attr(1, 1)
    wait_attr(0, 0)
    expand(0)
    issue_attr(2, 0)
    out_copy(0, 0).start()
    wait_attr(1, 1)
    expand(1)
    issue_attr(3, 1)
    out_copy(1, 1).start()

    def steady(t0, is_static_tail):
        wait_attr(t0, 0)
        out_copy(t0 - 2, 0).wait()
        expand(0)
        if not is_static_tail or t0 + 2 <= T_FULL - 1:
            issue_attr(t0 + 2, 0)
        out_copy(t0, 0).start()

        wait_attr(t0 + 1, 1)
        out_copy(t0 - 1, 1).wait()
        expand(1)
        if not is_static_tail or t0 + 3 <= T_FULL - 1:
            issue_attr(t0 + 3, 1)
        out_copy(t0 + 1, 1).start()

    def loop_body(j, carry):
        steady(j * 2, False)
        return carry

    lax.fori_loop(1, 18, loop_body, 0)

    steady(36, True)

    t = 38
    wait_attr(t, 0)
    out_copy(t - 2, 0).wait()
    expand(0)
    out_copy(t, 0).start()
    out_copy(t - 1, 1).wait()
    out_copy(t, 0).wait()

    @pl.when(wid < NLEFT)
    def _():
        t_extra = T_FULL
        issue_attr(t_extra, 0)
        wait_attr(t_extra, 0)
        expand(0)
        out_copy(t_extra, 0).start()
        out_copy(t_extra, 0).wait()


_sc_gather = functools.partial(
    pl.kernel,
    out_type=jax.ShapeDtypeStruct((E * EMB,), jnp.float32),
    mesh=plsc.VectorSubcoreMesh(core_axis_name="c", subcore_axis_name="s"),
    compiler_params=pltpu.CompilerParams(needs_layout_passes=False),
    scratch_types=[
        pltpu.VMEM((48 * EMB,), jnp.float32),
        pltpu.VMEM((2, SUPER), jnp.int32),
        pltpu.VMEM((2, SUPER), jnp.int32),
        pltpu.VMEM((2, SUPER), jnp.int32),
        pltpu.VMEM((SUPER * EMB,), jnp.float32),
        pltpu.VMEM((SUPER * EMB,), jnp.float32),
        pltpu.SemaphoreType.DMA,
        pltpu.SemaphoreType.DMA,
        pltpu.SemaphoreType.DMA,
        pltpu.SemaphoreType.DMA,
        pltpu.SemaphoreType.DMA,
    ],
)(_sc_body)


@jax.jit
def kernel(edge_attr, W0, W1, W2):
    a = edge_attr.astype(jnp.int32)
    table = _build_table(W0, W1, W2).reshape(-1)
    flat = _sc_gather(a[:, 0], a[:, 1], a[:, 2], table)
    return flat.reshape(E, EMB)


# speedup vs baseline: 14.0499x; 10.6651x over previous
import functools

import jax
import jax.numpy as jnp
from jax import lax
from jax.experimental import pallas as pl
from jax.experimental.pallas import tpu as pltpu
from jax.experimental.pallas import tpu_sc as plsc

EMB = 128
E = 320000
NW = 32
SUPER = 256
GROUPS = SUPER // 16
NSUPER = E // SUPER
T_FULL = NSUPER // NW
NLEFT = NSUPER - T_FULL * NW


def _table_body(w0_ref, w1_ref, w2_ref, t_ref):
    for i in range(4):
        for j in range(2):
            base = i * 12 + j * 6
            t_ref[base:base + 6, :] = (
                w2_ref[:, :] + w0_ref[i:i + 1, :] + w1_ref[j:j + 1, :]
            )


def _build_table(w0, w1, w2):
    return pl.pallas_call(
        _table_body,
        out_shape=jax.ShapeDtypeStruct((48, EMB), jnp.float32),
    )(w0, w1, w2)


def _sc_body(a0_hbm, a1_hbm, a2_hbm, table_hbm, out_hbm,
             table_v, a0_v, a1_v, a2_v, rows0_v, rows1_v,
             a_sem0, a_sem1, o_sem0, o_sem1, t_sem):
    wid = lax.axis_index("s") * 2 + lax.axis_index("c")
    a_sems = (a_sem0, a_sem1)
    o_sems = (o_sem0, o_sem1)
    rows = (rows0_v, rows1_v)

    def cid_of(t):
        return t * NW + wid

    def attr_copies(t, b):
        base = cid_of(t) * SUPER
        return [
            pltpu.make_async_copy(a0_hbm.at[pl.ds(base, SUPER)], a0_v.at[b], a_sems[b]),
            pltpu.make_async_copy(a1_hbm.at[pl.ds(base, SUPER)], a1_v.at[b], a_sems[b]),
            pltpu.make_async_copy(a2_hbm.at[pl.ds(base, SUPER)], a2_v.at[b], a_sems[b]),
        ]

    def issue_attr(t, b):
        for c in attr_copies(t, b):
            c.start()

    def wait_attr(t, b):
        for c in attr_copies(t, b):
            c.wait()

    def out_copy(t, b):
        base = cid_of(t) * SUPER * EMB
        return pltpu.make_async_copy(
            rows[b], out_hbm.at[pl.ds(base, SUPER * EMB)], o_sems[b])

    iota = lax.iota(jnp.int32, 16)
    iota128 = iota * EMB

    def expand(b):
        rb = rows[b]

        def group(g, carry):
            sl = pl.ds(g * 16, 16)
            code = (a0_v[b, sl] * 12 + a1_v[b, sl] * 6 + a2_v[b, sl]) * EMB
            dst0 = g * (16 * EMB)
            for r in range(16):
                src = code[r]
                dstb = dst0 + r * EMB
                vals = [table_v[pl.ds(src + s * 16, 16)] for s in range(8)]
                for s in range(8):
                    rb[pl.ds(dstb + s * 16, 16)] = vals[s]
            return carry

        lax.fori_loop(0, GROUPS, group, 0)

    pltpu.async_copy(table_hbm, table_v, t_sem).wait()

    issue_attr(0, 0)
    issue_attr(1, 1)
    wait_attr(0, 0)
    expand(0)
    issue_attr(2, 0)
    out_copy(0, 0).start()
    wait_attr(1, 1)
    expand(1)
    issue_attr(3, 1)
    out_copy(1, 1).start()

    def steady(t0, is_static_tail):
        wait_attr(t0, 0)
        out_copy(t0 - 2, 0).wait()
        expand(0)
        if not is_static_tail or t0 + 2 <= T_FULL - 1:
            issue_attr(t0 + 2, 0)
        out_copy(t0, 0).start()

        wait_attr(t0 + 1, 1)
        out_copy(t0 - 1, 1).wait()
        expand(1)
        if not is_static_tail or t0 + 3 <= T_FULL - 1:
            issue_attr(t0 + 3, 1)
        out_copy(t0 + 1, 1).start()

    def loop_body(j, carry):
        steady(j * 2, False)
        return carry

    lax.fori_loop(1, 18, loop_body, 0)

    steady(36, True)

    t = 38
    wait_attr(t, 0)
    out_copy(t - 2, 0).wait()
    expand(0)
    out_copy(t, 0).start()
    out_copy(t - 1, 1).wait()
    out_copy(t, 0).wait()

    @pl.when(wid < NLEFT)
    def _():
        t_extra = T_FULL
        issue_attr(t_extra, 0)
        wait_attr(t_extra, 0)
        expand(0)
        out_copy(t_extra, 0).start()
        out_copy(t_extra, 0).wait()


_sc_gather = functools.partial(
    pl.kernel,
    out_type=jax.ShapeDtypeStruct((E * EMB,), jnp.float32),
    mesh=plsc.VectorSubcoreMesh(core_axis_name="c", subcore_axis_name="s"),
    compiler_params=pltpu.CompilerParams(needs_layout_passes=False),
    scratch_types=[
        pltpu.VMEM((48 * EMB,), jnp.float32),
        pltpu.VMEM((2, SUPER), jnp.int32),
        pltpu.VMEM((2, SUPER), jnp.int32),
        pltpu.VMEM((2, SUPER), jnp.int32),
        pltpu.VMEM((SUPER * EMB,), jnp.float32),
        pltpu.VMEM((SUPER * EMB,), jnp.float32),
        pltpu.SemaphoreType.DMA,
        pltpu.SemaphoreType.DMA,
        pltpu.SemaphoreType.DMA,
        pltpu.SemaphoreType.DMA,
        pltpu.SemaphoreType.DMA,
    ],
)(_sc_body)


@jax.jit
def kernel(edge_attr, W0, W1, W2):
    a = edge_attr.astype(jnp.int32)
    table = _build_table(W0, W1, W2).reshape(-1)
    flat = _sc_gather(a[:, 0], a[:, 1], a[:, 2], table)
    return flat.reshape(E, EMB)
